# baseline (speedup 1.0000x reference)
import functools

import jax
import jax.numpy as jnp
from jax import lax
from jax.experimental import pallas as pl
from jax.experimental.pallas import tpu as pltpu
from jax.experimental.pallas import tpu_sc as plsc


@functools.lru_cache(maxsize=None)
def _make_gather(V, D, B0, B1):
    info = plsc.get_sparse_core_info()
    NC, NS, L = info.num_cores, info.num_subcores, info.num_lanes
    NW = NC * NS
    assert B0 % NW == 0 and D == 64
    G = B0 // NW
    assert G == 128
    NBUF = 4
    assert B1 % 2 == 0 and B1 >= 6
    n_steady = (B1 - 2) // NBUF
    n_tail = B1 - NBUF * n_steady
    mesh = plsc.VectorSubcoreMesh(core_axis_name="c", subcore_axis_name="s")

    DT, DS = D // 8, 8

    @functools.partial(
        pl.kernel,
        mesh=mesh,
        out_type=jax.ShapeDtypeStruct((B1, DT, NW, DS, G), jnp.float32),
        scratch_types=[
            pltpu.VMEM((B1 * G,), jnp.int32),
            [pltpu.VMEM((G, D), jnp.float32) for _ in range(NBUF)],
            [pltpu.VMEM((1, DT, 1, DS, G), jnp.float32) for _ in range(2)],
            [pltpu.SemaphoreType.DMA for _ in range(NBUF)],
            [pltpu.SemaphoreType.DMA for _ in range(2)],
        ],
        compiler_params=pltpu.CompilerParams(
            use_tc_tiling_on_sc=True, needs_layout_passes=False
        ),
    )
    def k(table_hbm, idx_hbm, out_hbm, idx_v, stag, blk, sem_g, sem_s):
        wid = lax.axis_index("s") * NC + lax.axis_index("c")
        base = wid * (B1 * G)
        pltpu.sync_copy(idx_hbm.at[pl.ds(base, B1 * G)], idx_v)
        lanes = lax.iota(jnp.int32, L)
        zeros = jnp.zeros((L,), jnp.int32)
        dt16 = [(v * L + lanes) // DS for v in range(D // L)]
        ds16 = [(v * L + lanes) % DS for v in range(D // L)]

        def start_gather(b1, b):
            def grp(g, _):
                pos16 = (g * L + lanes) * B1 + b1
                i16 = plsc.load_gather(idx_v, [pos16])
                for l in range(L):
                    pltpu.async_copy(
                        table_hbm.at[pl.ds(i16[l], 1)],
                        stag[b].at[pl.ds(g * L + l, 1)],
                        sem_g[b],
                    )
                return 0

            lax.fori_loop(0, G // L, grp, 0)

        def drain_gather(b):
            def w(p, _):
                pltpu.make_async_copy(
                    table_hbm.at[pl.ds(0, 1)], stag[b].at[pl.ds(0, 1)], sem_g[b]
                ).wait()
                return 0

            lax.fori_loop(0, G, w, 0)

        def permute(b, o):
            def row(q, _):
                q16 = zeros + q
                for v in range(D // L):
                    sv = stag[b][q, pl.ds(v * L, L)]
                    plsc.store_scatter(
                        blk[o], [zeros, dt16[v], zeros, ds16[v], q16], sv
                    )
                return 0

            lax.fori_loop(0, G, row, 0)

        def start_store(b1, o):
            return pltpu.async_copy(
                blk[o],
                out_hbm.at[pl.ds(b1, 1), :, pl.ds(wid, 1), :, :],
                sem_s[o],
            )

        def wait_store(o):
            pltpu.make_async_copy(
                blk[o],
                out_hbm.at[pl.ds(0, 1), :, pl.ds(0, 1), :, :],
                sem_s[o],
            ).wait()

        for b in range(NBUF):
            start_gather(b, b)

        def process(b1, b, o, first, last):
            drain_gather(b)
            nxt = b1 + NBUF
            if isinstance(b1, int):
                if nxt < B1:
                    start_gather(nxt, b)
            else:

                @pl.when(nxt < B1)
                def _():
                    start_gather(nxt, b)

            if not first:
                if isinstance(b1, int):
                    wait_store(o)
                else:

                    @pl.when(b1 >= 2)
                    def _():
                        wait_store(o)

            permute(b, o)
            start_store(b1, o)

        def steady(j, _):
            for i in range(NBUF):
                b1 = NBUF * j + i
                process(b1, i, i % 2, first=False, last=False)
            return 0

        lax.fori_loop(0, n_steady, steady, 0)
        for t in range(n_tail):
            b1 = NBUF * n_steady + t
            process(b1, b1 % NBUF, b1 % 2, first=False, last=True)
        for o in range(2):
            wait_store(o)

    return k


def kernel(input_, weight):
    B0, B1 = input_.shape
    V, D = weight.shape
    idx = input_.reshape(B0 * B1).astype(jnp.int32)
    x2 = _make_gather(V, D, B0, B1)(weight, idx)
    return x2.transpose(2, 4, 0, 1, 3).reshape(B0, B1, D)


# speedup vs baseline: 1.0502x; 1.0502x over previous
import functools

import jax
import jax.numpy as jnp
from jax import lax
from jax.experimental import pallas as pl
from jax.experimental.pallas import tpu as pltpu
from jax.experimental.pallas import tpu_sc as plsc


@functools.lru_cache(maxsize=None)
def _make_gather(V, D, B0, B1):
    info = plsc.get_sparse_core_info()
    NC, NS, L = info.num_cores, info.num_subcores, info.num_lanes
    NW = NC * NS
    assert B0 % NW == 0 and D == 64
    G = B0 // NW
    assert G == 128
    NBUF = 4
    assert B1 % 2 == 0 and B1 >= 6
    n_steady = (B1 - 2) // NBUF
    n_tail = B1 - NBUF * n_steady
    mesh = plsc.VectorSubcoreMesh(core_axis_name="c", subcore_axis_name="s")

    DT, DS = D // 8, 8

    @functools.partial(
        pl.kernel,
        mesh=mesh,
        out_type=jax.ShapeDtypeStruct((B1, DT, NW, DS, G), jnp.float32),
        scratch_types=[
            pltpu.VMEM((B1 * G,), jnp.int32),
            [pltpu.VMEM((G, D), jnp.float32) for _ in range(NBUF)],
            [pltpu.VMEM((1, DT, 1, DS, G), jnp.float32) for _ in range(2)],
            [pltpu.SemaphoreType.DMA for _ in range(NBUF)],
            [pltpu.SemaphoreType.DMA for _ in range(2)],
        ],
        compiler_params=pltpu.CompilerParams(
            use_tc_tiling_on_sc=True, needs_layout_passes=False
        ),
    )
    def k(table_hbm, idx_hbm, out_hbm, idx_v, stag, blk, sem_g, sem_s):
        wid = lax.axis_index("s") * NC + lax.axis_index("c")
        base = wid * (B1 * G)
        pltpu.sync_copy(idx_hbm.at[pl.ds(base, B1 * G)], idx_v)
        lanes = lax.iota(jnp.int32, L)
        zeros = jnp.zeros((L,), jnp.int32)
        dt16 = [(v * L + lanes) // DS for v in range(D // L)]
        ds16 = [(v * L + lanes) % DS for v in range(D // L)]

        def start_gather(b1, b):
            def grp(g, _):
                pos16 = (g * L + lanes) * B1 + b1
                i16 = plsc.load_gather(idx_v, [pos16])
                for l in range(L):
                    pltpu.async_copy(
                        table_hbm.at[pl.ds(i16[l], 1)],
                        stag[b].at[pl.ds(g * L + l, 1)],
                        sem_g[b],
                    )
                return 0

            lax.fori_loop(0, G // L, grp, 0)

        def drain_gather(b):
            pltpu.make_async_copy(
                table_hbm.at[pl.ds(0, G)], stag[b], sem_g[b]
            ).wait()

        def permute(b, o):
            def row(q, _):
                q16 = zeros + q
                for v in range(D // L):
                    sv = stag[b][q, pl.ds(v * L, L)]
                    plsc.store_scatter(
                        blk[o], [zeros, dt16[v], zeros, ds16[v], q16], sv
                    )
                return 0

            lax.fori_loop(0, G, row, 0)

        def start_store(b1, o):
            return pltpu.async_copy(
                blk[o],
                out_hbm.at[pl.ds(b1, 1), :, pl.ds(wid, 1), :, :],
                sem_s[o],
            )

        def wait_store(o):
            pltpu.make_async_copy(
                blk[o],
                out_hbm.at[pl.ds(0, 1), :, pl.ds(0, 1), :, :],
                sem_s[o],
            ).wait()

        for b in range(NBUF):
            start_gather(b, b)

        def process(b1, b, o, first, last):
            drain_gather(b)
            if not first:
                if isinstance(b1, int):
                    wait_store(o)
                else:

                    @pl.when(b1 >= 2)
                    def _():
                        wait_store(o)

            permute(b, o)
            start_store(b1, o)
            nxt = b1 + NBUF
            if isinstance(b1, int):
                if nxt < B1:
                    start_gather(nxt, b)
            else:

                @pl.when(nxt < B1)
                def _():
                    start_gather(nxt, b)

        def steady(j, _):
            for i in range(NBUF):
                b1 = NBUF * j + i
                process(b1, i, i % 2, first=False, last=False)
            return 0

        lax.fori_loop(0, n_steady, steady, 0)
        for t in range(n_tail):
            b1 = NBUF * n_steady + t
            process(b1, b1 % NBUF, b1 % 2, first=False, last=True)
        for o in range(2):
            wait_store(o)

    return k


def kernel(input_, weight):
    B0, B1 = input_.shape
    V, D = weight.shape
    idx = input_.reshape(B0 * B1).astype(jnp.int32)
    x2 = _make_gather(V, D, B0, B1)(weight, idx)
    return x2.transpose(2, 4, 0, 1, 3).reshape(B0, B1, D)


# speedup vs baseline: 1.1232x; 1.0695x over previous
import functools

import jax
import jax.numpy as jnp
from jax import lax
from jax.experimental import pallas as pl
from jax.experimental.pallas import tpu as pltpu
from jax.experimental.pallas import tpu_sc as plsc


@functools.lru_cache(maxsize=None)
def _make_gather(V, D, B):
    info = plsc.get_sparse_core_info()
    NC, NS, L = info.num_cores, info.num_subcores, info.num_lanes
    NW = NC * NS
    assert B % NW == 0
    b_per_w = B // NW
    C = 160
    NBUF = 4
    assert b_per_w % C == 0 and C % L == 0
    n_chunks = b_per_w // C
    assert n_chunks >= 2 * NBUF
    mesh = plsc.VectorSubcoreMesh(core_axis_name="c", subcore_axis_name="s")

    @functools.partial(
        pl.kernel,
        mesh=mesh,
        out_type=jax.ShapeDtypeStruct((B, D), jnp.float32),
        scratch_types=[
            pltpu.VMEM((b_per_w,), jnp.int32),
            [pltpu.VMEM((C, D), jnp.float32) for _ in range(NBUF)],
            [pltpu.SemaphoreType.DMA for _ in range(NBUF)],
            [pltpu.SemaphoreType.DMA for _ in range(NBUF)],
        ],
        compiler_params=pltpu.CompilerParams(use_tc_tiling_on_sc=True),
    )
    def k(table_hbm, idx_hbm, out_hbm, idx_v, rows, sem_g, sem_s):
        wid = lax.axis_index("s") * NC + lax.axis_index("c")
        base = wid * b_per_w
        pltpu.sync_copy(idx_hbm.at[pl.ds(base, b_per_w)], idx_v)

        def start_gather(chunk, b):
            def grp(g, _):
                i16 = idx_v[pl.ds(chunk * C + g * L, L)]
                for l in range(L):
                    pltpu.async_copy(
                        table_hbm.at[pl.ds(i16[l], 1)],
                        rows[b].at[pl.ds(g * L + l, 1)],
                        sem_g[b],
                    )
                return 0

            lax.fori_loop(0, C // L, grp, 0)

        def drain_gather(b):
            pltpu.make_async_copy(
                table_hbm.at[pl.ds(0, C)], rows[b], sem_g[b]
            ).wait()

        def start_store(chunk, b):
            return pltpu.async_copy(
                rows[b], out_hbm.at[pl.ds(base + chunk * C, C)], sem_s[b]
            )

        for b in range(NBUF):
            start_gather(b, b)
        stores = [None] * NBUF
        for i in range(n_chunks):
            b = i % NBUF
            drain_gather(b)
            stores[b] = start_store(i, b)
            nxt = i + NBUF
            if nxt < n_chunks:
                stores[b].wait()
                start_gather(nxt, b)
        for b in range(NBUF):
            stores[b].wait()

    return k


def kernel(input_, weight):
    S0, S1 = input_.shape
    B = S0 * S1
    V, D = weight.shape
    idx = input_.reshape(B).astype(jnp.int32)
    out = _make_gather(V, D, B)(weight, idx)
    return out.reshape(S0, S1, D)
